# baseline (speedup 1.0000x reference)
import functools

import jax
import jax.numpy as jnp
from jax.experimental import pallas as pl
from jax.experimental.pallas import tpu as pltpu

_NUM_EMBEDDINGS = 8192
_EMBED_DIM = 32
_SPARSITY = 5
_COMMIT = 0.25
_EPS = 1e-10
_TB = 512


def _fast_marked(corr):
    a = jnp.abs(corr)
    m = jnp.max(a, axis=1, keepdims=True)
    for _ in range(_SPARSITY - 1):
        m = jnp.max(jnp.where(a >= m, -1.0, a), axis=1, keepdims=True)
    return a >= m


def _exact_marked(corr):
    a = jnp.abs(corr)
    iota = jax.lax.broadcasted_iota(jnp.int32, a.shape, 1).astype(jnp.float32)
    for _ in range(_SPARSITY):
        m = jnp.max(a, axis=1, keepdims=True)
        idx = jnp.min(jnp.where(a == m, iota, float(_NUM_EMBEDDINGS)),
                      axis=1, keepdims=True)
        a = jnp.where(iota == idx, -1.0, a)
    return a < 0.0


def _body(marks_fn, xp_ref, x_ref, d_ref, q_ref, loss_ref, bad_ref,
          corr_ref, dn_ref):
    i = pl.program_id(0)
    nb = pl.num_programs(0)

    @pl.when(i == 0)
    def _prologue():
        D = d_ref[...]
        norm = jnp.sqrt(jnp.sum(D * D, axis=0, keepdims=True))
        dn_ref[...] = D / (norm + _EPS)
        loss_ref[...] = jnp.zeros_like(loss_ref)
        bad_ref[...] = jnp.zeros_like(bad_ref)
        corr_ref[0] = jnp.dot(x_ref[...], dn_ref[...])

    Dn = dn_ref[...]

    @pl.when(i + 1 < nb)
    def _prefetch():
        corr_ref[(i + 1) % 2] = jnp.dot(xp_ref[...], Dn)

    corr = corr_ref[i % 2]
    marked = marks_fn(corr)
    cnt = jnp.dot(jnp.where(marked, 1.0, 0.0).astype(jnp.bfloat16),
                  jnp.ones((_NUM_EMBEDDINGS, 8), jnp.bfloat16),
                  preferred_element_type=jnp.float32)
    marks = jnp.sum(cnt) * 0.125
    bad_ref[...] += (marks - float(_SPARSITY * _TB)).reshape(1, 1)
    coef = jnp.where(marked, corr, 0.0)
    recon = jnp.dot(coef, Dn.T)
    x = x_ref[...]
    diff = recon - x
    q_ref[...] = x + diff
    loss_ref[...] += jnp.sum(diff * diff).reshape(1, 1)


def _run(marks_fn, X, dictionary, Bt):
    nb = Bt // _TB
    return pl.pallas_call(
        functools.partial(_body, marks_fn),
        grid=(nb,),
        in_specs=[
            pl.BlockSpec((_TB, _EMBED_DIM), lambda i: ((i + 1) % nb, 0)),
            pl.BlockSpec((_TB, _EMBED_DIM), lambda i: (i, 0)),
            pl.BlockSpec((_EMBED_DIM, _NUM_EMBEDDINGS), lambda i: (0, 0)),
        ],
        out_specs=[
            pl.BlockSpec((_TB, _EMBED_DIM), lambda i: (i, 0)),
            pl.BlockSpec((1, 1), lambda i: (0, 0)),
            pl.BlockSpec((1, 1), lambda i: (0, 0)),
        ],
        out_shape=[
            jax.ShapeDtypeStruct((Bt, _EMBED_DIM), jnp.float32),
            jax.ShapeDtypeStruct((1, 1), jnp.float32),
            jax.ShapeDtypeStruct((1, 1), jnp.float32),
        ],
        scratch_shapes=[
            pltpu.VMEM((2, _TB, _NUM_EMBEDDINGS), jnp.float32),
            pltpu.VMEM((_EMBED_DIM, _NUM_EMBEDDINGS), jnp.float32),
        ],
    )(X, X, dictionary)


def kernel(z, dictionary):
    B, T, C = z.shape
    X = z.reshape(-1, C)
    Bt = X.shape[0]
    q, losssum, bad = _run(_fast_marked, X, dictionary, Bt)
    q, losssum = jax.lax.cond(
        bad[0, 0] > 0.5,
        lambda: _run(_exact_marked, X, dictionary, Bt)[:2],
        lambda: (q, losssum),
    )
    loss = (1.0 + _COMMIT) * losssum[0, 0] / (Bt * C)
    return q.reshape(B, T, C), loss


# speedup vs baseline: 1.0205x; 1.0205x over previous
import functools

import jax
import jax.numpy as jnp
from jax.experimental import pallas as pl
from jax.experimental.pallas import tpu as pltpu

_NUM_EMBEDDINGS = 8192
_EMBED_DIM = 32
_SPARSITY = 5
_COMMIT = 0.25
_EPS = 1e-10
_TB = 256


def _thresholds(a, k):
    ms = [jnp.max(a, axis=1, keepdims=True)]
    for _ in range(k - 1):
        ms.append(jnp.max(jnp.where(a >= ms[-1], -1.0, a),
                          axis=1, keepdims=True))
    return ms


def _fast_marked(corr):
    a = jnp.abs(corr)
    n2 = _NUM_EMBEDDINGS // 2
    left, right = a[:, :n2], a[:, n2:]
    hi = jnp.maximum(left, right)
    lo = jnp.minimum(left, right)
    cand = jnp.concatenate(
        _thresholds(hi, _SPARSITY) + _thresholds(lo, 2), axis=1)
    m = _thresholds(cand, _SPARSITY)[-1]
    return a >= m


def _exact_marked(corr):
    a = jnp.abs(corr)
    iota = jax.lax.broadcasted_iota(jnp.int32, a.shape, 1).astype(jnp.float32)
    for _ in range(_SPARSITY):
        m = jnp.max(a, axis=1, keepdims=True)
        idx = jnp.min(jnp.where(a == m, iota, float(_NUM_EMBEDDINGS)),
                      axis=1, keepdims=True)
        a = jnp.where(iota == idx, -1.0, a)
    return a < 0.0


def _body(marks_fn, xp_ref, x_ref, d_ref, q_ref, loss_ref, bad_ref,
          corr_ref, dn_ref):
    i = pl.program_id(0)
    nb = pl.num_programs(0)

    @pl.when(i == 0)
    def _prologue():
        D = d_ref[...]
        norm = jnp.sqrt(jnp.sum(D * D, axis=0, keepdims=True))
        dn_ref[...] = D / (norm + _EPS)
        loss_ref[...] = jnp.zeros_like(loss_ref)
        bad_ref[...] = jnp.zeros_like(bad_ref)
        corr_ref[0] = jnp.dot(x_ref[...], dn_ref[...])

    Dn = dn_ref[...]

    @pl.when(i + 1 < nb)
    def _prefetch():
        corr_ref[(i + 1) % 2] = jnp.dot(xp_ref[...], Dn)

    corr = corr_ref[i % 2]
    marked = marks_fn(corr)
    cnt = jnp.dot(jnp.where(marked, 1.0, 0.0).astype(jnp.bfloat16),
                  jnp.ones((_NUM_EMBEDDINGS, 8), jnp.bfloat16),
                  preferred_element_type=jnp.float32)
    marks = jnp.sum(cnt) * 0.125
    bad_ref[...] += (marks - float(_SPARSITY * _TB)).reshape(1, 1)
    coef = jnp.where(marked, corr, 0.0)
    recon = jnp.dot(coef, Dn.T)
    x = x_ref[...]
    diff = recon - x
    q_ref[...] = x + diff
    loss_ref[...] += jnp.sum(diff * diff).reshape(1, 1)


def _run(marks_fn, X, dictionary, Bt):
    nb = Bt // _TB
    return pl.pallas_call(
        functools.partial(_body, marks_fn),
        grid=(nb,),
        in_specs=[
            pl.BlockSpec((_TB, _EMBED_DIM), lambda i: ((i + 1) % nb, 0)),
            pl.BlockSpec((_TB, _EMBED_DIM), lambda i: (i, 0)),
            pl.BlockSpec((_EMBED_DIM, _NUM_EMBEDDINGS), lambda i: (0, 0)),
        ],
        out_specs=[
            pl.BlockSpec((_TB, _EMBED_DIM), lambda i: (i, 0)),
            pl.BlockSpec((1, 1), lambda i: (0, 0)),
            pl.BlockSpec((1, 1), lambda i: (0, 0)),
        ],
        out_shape=[
            jax.ShapeDtypeStruct((Bt, _EMBED_DIM), jnp.float32),
            jax.ShapeDtypeStruct((1, 1), jnp.float32),
            jax.ShapeDtypeStruct((1, 1), jnp.float32),
        ],
        scratch_shapes=[
            pltpu.VMEM((2, _TB, _NUM_EMBEDDINGS), jnp.float32),
            pltpu.VMEM((_EMBED_DIM, _NUM_EMBEDDINGS), jnp.float32),
        ],
    )(X, X, dictionary)


def kernel(z, dictionary):
    B, T, C = z.shape
    X = z.reshape(-1, C)
    Bt = X.shape[0]
    q, losssum, bad = _run(_fast_marked, X, dictionary, Bt)
    q, losssum = jax.lax.cond(
        bad[0, 0] > 0.5,
        lambda: _run(_exact_marked, X, dictionary, Bt)[:2],
        lambda: (q, losssum),
    )
    loss = (1.0 + _COMMIT) * losssum[0, 0] / (Bt * C)
    return q.reshape(B, T, C), loss


# speedup vs baseline: 1.0536x; 1.0324x over previous
import functools

import jax
import jax.numpy as jnp
from jax.experimental import pallas as pl
from jax.experimental.pallas import tpu as pltpu

_NUM_EMBEDDINGS = 8192
_EMBED_DIM = 32
_SPARSITY = 5
_COMMIT = 0.25
_EPS = 1e-10
_TB = 384


def _thresholds(a, k):
    ms = [jnp.max(a, axis=1, keepdims=True)]
    for _ in range(k - 1):
        ms.append(jnp.max(jnp.where(a >= ms[-1], -1.0, a),
                          axis=1, keepdims=True))
    return ms


def _fast_marked(corr):
    n2 = _NUM_EMBEDDINGS // 2
    left = jnp.abs(corr[:, :n2])
    right = jnp.abs(corr[:, n2:])
    hi = jnp.maximum(left, right)
    lo = jnp.minimum(left, right)
    cand = jnp.concatenate(
        _thresholds(hi, _SPARSITY) + _thresholds(lo, 2), axis=1)
    m = _thresholds(cand, _SPARSITY)[-1]
    return jnp.abs(corr) >= m


def _exact_marked(corr):
    a = jnp.abs(corr)
    iota = jax.lax.broadcasted_iota(jnp.int32, a.shape, 1).astype(jnp.float32)
    for _ in range(_SPARSITY):
        m = jnp.max(a, axis=1, keepdims=True)
        idx = jnp.min(jnp.where(a == m, iota, float(_NUM_EMBEDDINGS)),
                      axis=1, keepdims=True)
        a = jnp.where(iota == idx, -1.0, a)
    return a < 0.0


def _body(marks_fn, xp_ref, x_ref, d_ref, q_ref, loss_ref, bad_ref,
          corr_ref, dn_ref):
    i = pl.program_id(0)
    nb = pl.num_programs(0)

    @pl.when(i == 0)
    def _prologue():
        D = d_ref[...]
        norm = jnp.sqrt(jnp.sum(D * D, axis=0, keepdims=True))
        dn_ref[...] = D / (norm + _EPS)
        loss_ref[...] = jnp.zeros_like(loss_ref)
        bad_ref[...] = jnp.zeros_like(bad_ref)
        corr_ref[0] = jnp.dot(x_ref[...], dn_ref[...])

    Dn = dn_ref[...]

    @pl.when(i + 1 < nb)
    def _prefetch():
        corr_ref[(i + 1) % 2] = jnp.dot(xp_ref[...], Dn)

    corr = corr_ref[i % 2]
    marked = marks_fn(corr)
    cnt = jnp.dot(jnp.where(marked, 1.0, 0.0).astype(jnp.bfloat16),
                  jnp.ones((_NUM_EMBEDDINGS, 8), jnp.bfloat16),
                  preferred_element_type=jnp.float32)
    marks = jnp.sum(cnt) * 0.125
    bad_ref[...] += (marks - float(_SPARSITY * _TB)).reshape(1, 1)
    coef = jnp.where(marked, corr, 0.0)
    recon = jnp.dot(coef, Dn.T)
    x = x_ref[...]
    diff = recon - x
    q_ref[...] = x + diff
    loss_ref[...] += jnp.sum(diff * diff).reshape(1, 1)


def _run(marks_fn, X, dictionary, Bt):
    nb = Bt // _TB
    return pl.pallas_call(
        functools.partial(_body, marks_fn),
        grid=(nb,),
        in_specs=[
            pl.BlockSpec((_TB, _EMBED_DIM), lambda i: ((i + 1) % nb, 0)),
            pl.BlockSpec((_TB, _EMBED_DIM), lambda i: (i, 0)),
            pl.BlockSpec((_EMBED_DIM, _NUM_EMBEDDINGS), lambda i: (0, 0)),
        ],
        out_specs=[
            pl.BlockSpec((_TB, _EMBED_DIM), lambda i: (i, 0)),
            pl.BlockSpec((1, 1), lambda i: (0, 0)),
            pl.BlockSpec((1, 1), lambda i: (0, 0)),
        ],
        out_shape=[
            jax.ShapeDtypeStruct((Bt, _EMBED_DIM), jnp.float32),
            jax.ShapeDtypeStruct((1, 1), jnp.float32),
            jax.ShapeDtypeStruct((1, 1), jnp.float32),
        ],
        scratch_shapes=[
            pltpu.VMEM((2, _TB, _NUM_EMBEDDINGS), jnp.float32),
            pltpu.VMEM((_EMBED_DIM, _NUM_EMBEDDINGS), jnp.float32),
        ],
    )(X, X, dictionary)


def kernel(z, dictionary):
    B, T, C = z.shape
    X = z.reshape(-1, C)
    Bt = X.shape[0]
    q, losssum, bad = _run(_fast_marked, X, dictionary, Bt)
    q, losssum = jax.lax.cond(
        bad[0, 0] > 0.5,
        lambda: _run(_exact_marked, X, dictionary, Bt)[:2],
        lambda: (q, losssum),
    )
    loss = (1.0 + _COMMIT) * losssum[0, 0] / (Bt * C)
    return q.reshape(B, T, C), loss


# speedup vs baseline: 1.1043x; 1.0481x over previous
import functools

import jax
import jax.numpy as jnp
from jax.experimental import pallas as pl
from jax.experimental.pallas import tpu as pltpu

_NUM_EMBEDDINGS = 8192
_EMBED_DIM = 32
_SPARSITY = 5
_COMMIT = 0.25
_EPS = 1e-10
_TB = 384


def _thresholds(a, k):
    ms = [jnp.max(a, axis=1, keepdims=True)]
    for _ in range(k - 1):
        ms.append(jnp.max(jnp.where(a >= ms[-1], -1.0, a),
                          axis=1, keepdims=True))
    return ms


def _fast_marked(corr):
    n4 = _NUM_EMBEDDINGS // 4
    q1 = jnp.abs(corr[:, :n4])
    q2 = jnp.abs(corr[:, n4:2 * n4])
    q3 = jnp.abs(corr[:, 2 * n4:3 * n4])
    q4 = jnp.abs(corr[:, 3 * n4:])
    hi1, lo1 = jnp.maximum(q1, q2), jnp.minimum(q1, q2)
    hi2, lo2 = jnp.maximum(q3, q4), jnp.minimum(q3, q4)
    hh, hl = jnp.maximum(hi1, hi2), jnp.minimum(hi1, hi2)
    lh, ll = jnp.maximum(lo1, lo2), jnp.minimum(lo1, lo2)
    cand = jnp.concatenate(
        _thresholds(hh, _SPARSITY) + _thresholds(hl, 2)
        + _thresholds(lh, 2) + _thresholds(ll, 1), axis=1)
    m = _thresholds(cand, _SPARSITY)[-1]
    return jnp.abs(corr) >= m


def _exact_marked(corr):
    a = jnp.abs(corr)
    iota = jax.lax.broadcasted_iota(jnp.int32, a.shape, 1).astype(jnp.float32)
    for _ in range(_SPARSITY):
        m = jnp.max(a, axis=1, keepdims=True)
        idx = jnp.min(jnp.where(a == m, iota, float(_NUM_EMBEDDINGS)),
                      axis=1, keepdims=True)
        a = jnp.where(iota == idx, -1.0, a)
    return a < 0.0


def _body(marks_fn, xp_ref, x_ref, d_ref, q_ref, loss_ref, bad_ref,
          corr_ref, dn_ref):
    i = pl.program_id(0)
    nb = pl.num_programs(0)

    @pl.when(i == 0)
    def _prologue():
        D = d_ref[...]
        norm = jnp.sqrt(jnp.sum(D * D, axis=0, keepdims=True))
        dn_ref[...] = D / (norm + _EPS)
        loss_ref[...] = jnp.zeros_like(loss_ref)
        bad_ref[...] = jnp.zeros_like(bad_ref)
        corr_ref[0] = jnp.dot(x_ref[...], dn_ref[...])

    Dn = dn_ref[...]

    @pl.when(i + 1 < nb)
    def _prefetch():
        corr_ref[(i + 1) % 2] = jnp.dot(xp_ref[...], Dn)

    corr = corr_ref[i % 2]
    marked = marks_fn(corr)
    cnt = jnp.dot(jnp.where(marked, 1.0, 0.0).astype(jnp.bfloat16),
                  jnp.ones((_NUM_EMBEDDINGS, 8), jnp.bfloat16),
                  preferred_element_type=jnp.float32)
    marks = jnp.sum(cnt) * 0.125
    bad_ref[...] += (marks - float(_SPARSITY * _TB)).reshape(1, 1)
    coef = jnp.where(marked, corr, 0.0)
    recon = jnp.dot(coef, Dn.T)
    x = x_ref[...]
    diff = recon - x
    q_ref[...] = x + diff
    loss_ref[...] += jnp.sum(diff * diff).reshape(1, 1)


def _run(marks_fn, X, dictionary, Bt):
    nb = Bt // _TB
    return pl.pallas_call(
        functools.partial(_body, marks_fn),
        grid=(nb,),
        in_specs=[
            pl.BlockSpec((_TB, _EMBED_DIM), lambda i: ((i + 1) % nb, 0)),
            pl.BlockSpec((_TB, _EMBED_DIM), lambda i: (i, 0)),
            pl.BlockSpec((_EMBED_DIM, _NUM_EMBEDDINGS), lambda i: (0, 0)),
        ],
        out_specs=[
            pl.BlockSpec((_TB, _EMBED_DIM), lambda i: (i, 0)),
            pl.BlockSpec((1, 1), lambda i: (0, 0)),
            pl.BlockSpec((1, 1), lambda i: (0, 0)),
        ],
        out_shape=[
            jax.ShapeDtypeStruct((Bt, _EMBED_DIM), jnp.float32),
            jax.ShapeDtypeStruct((1, 1), jnp.float32),
            jax.ShapeDtypeStruct((1, 1), jnp.float32),
        ],
        scratch_shapes=[
            pltpu.VMEM((2, _TB, _NUM_EMBEDDINGS), jnp.float32),
            pltpu.VMEM((_EMBED_DIM, _NUM_EMBEDDINGS), jnp.float32),
        ],
    )(X, X, dictionary)


def kernel(z, dictionary):
    B, T, C = z.shape
    X = z.reshape(-1, C)
    Bt = X.shape[0]
    q, losssum, bad = _run(_fast_marked, X, dictionary, Bt)
    q, losssum = jax.lax.cond(
        bad[0, 0] > 0.5,
        lambda: _run(_exact_marked, X, dictionary, Bt)[:2],
        lambda: (q, losssum),
    )
    loss = (1.0 + _COMMIT) * losssum[0, 0] / (Bt * C)
    return q.reshape(B, T, C), loss
